# Initial kernel scaffold; baseline (speedup 1.0000x reference)
import jax
import jax.numpy as jnp
from jax.experimental import pallas as pl


def kernel(src, tgt, src_coords, tgt_coords):
    raise NotImplementedError("write your pallas kernel here")


# speedup vs baseline: 9.2159x; 9.2159x over previous
import jax
import jax.numpy as jnp
from jax.experimental import pallas as pl

N = 4096
C = 256
K = 16
R = 256
NBLK = N // R
NEG = float("-inf")
POS = float("inf")


def _body(coords_blk, coords_t, ncols, feats_blk, out_ref):
    x = coords_blk[0]
    xt = coords_t[0]
    f = feats_blk[0]

    x0 = x[:, 0:1]
    x1 = x[:, 1:2]
    x2 = x[:, 2:3]
    y0 = xt[0:1, :]
    y1 = xt[1:2, :]
    y2 = xt[2:3, :]

    aa_i = (x0 * x0 + x1 * x1) + x2 * x2
    aa_n = (y0 * y0 + y1 * y1) + y2 * y2
    xb0 = x0.astype(jnp.bfloat16).astype(jnp.float32)
    xb1 = x1.astype(jnp.bfloat16).astype(jnp.float32)
    xb2 = x2.astype(jnp.bfloat16).astype(jnp.float32)
    yb0 = y0.astype(jnp.bfloat16).astype(jnp.float32)
    yb1 = y1.astype(jnp.bfloat16).astype(jnp.float32)
    yb2 = y2.astype(jnp.bfloat16).astype(jnp.float32)
    ab = (xb0 * yb0 + xb1 * yb1) + xb2 * yb2
    cur = (aa_i + aa_n) - 2.0 * ab

    iota = jax.lax.broadcasted_iota(jnp.int32, (R, N), 1)
    m = jnp.full((R, 1), NEG, dtype=jnp.float32)
    for r in range(K + 1):
        mn = jnp.min(cur, axis=1, keepdims=True)
        sel = jnp.min(jnp.where(cur == mn, iota, N), axis=1, keepdims=True)
        onehot = iota == sel
        if r >= 1:
            col = ncols[0, r - 1 : r, :]
            val = jnp.max(jnp.where(onehot, col, NEG), axis=1, keepdims=True)
            m = jnp.maximum(m, val)
        cur = jnp.where(onehot, POS, cur)

    out_ref[0, :, :C] = f
    out_ref[0, :, C:] = m - f


@jax.jit
def _run(feats2, coords2, coords_t2, ncols2):
    return pl.pallas_call(
        _body,
        grid=(2, NBLK),
        in_specs=[
            pl.BlockSpec((1, R, 3), lambda t, b: (t, b, 0)),
            pl.BlockSpec((1, 3, N), lambda t, b: (t, 0, 0)),
            pl.BlockSpec((1, K, N), lambda t, b: (t, 0, 0)),
            pl.BlockSpec((1, R, C), lambda t, b: (t, b, 0)),
        ],
        out_specs=pl.BlockSpec((1, R, 2 * C), lambda t, b: (t, b, 0)),
        out_shape=jax.ShapeDtypeStruct((2, N, 2 * C), jnp.float32),
    )(coords2, coords_t2, ncols2, feats2)


def kernel(src, tgt, src_coords, tgt_coords):
    feats2 = jnp.stack([src, tgt])
    coords2 = jnp.stack([src_coords, tgt_coords])
    coords_t2 = jnp.transpose(coords2, (0, 2, 1))
    ncols2 = jnp.transpose(feats2[:, :, :K], (0, 2, 1))
    out = _run(feats2, coords2, coords_t2, ncols2)
    return out[0], out[1]


# speedup vs baseline: 15.2853x; 1.6586x over previous
import jax
import jax.numpy as jnp
from jax.experimental import pallas as pl

N = 4096
C = 256
K = 16
R = 256
NBLK = N // R
NEG = float("-inf")
POS = float("inf")


def _body(coords_blk, coords_t, ncols, feats_blk, out_ref):
    x = coords_blk[0]
    xt = coords_t[0]
    f = feats_blk[0]

    x0 = x[:, 0:1]
    x1 = x[:, 1:2]
    x2 = x[:, 2:3]
    y0 = xt[0:1, :]
    y1 = xt[1:2, :]
    y2 = xt[2:3, :]

    aa_i = (x0 * x0 + x1 * x1) + x2 * x2
    aa_n = (y0 * y0 + y1 * y1) + y2 * y2
    xb0 = x0.astype(jnp.bfloat16).astype(jnp.float32)
    xb1 = x1.astype(jnp.bfloat16).astype(jnp.float32)
    xb2 = x2.astype(jnp.bfloat16).astype(jnp.float32)
    yb0 = y0.astype(jnp.bfloat16).astype(jnp.float32)
    yb1 = y1.astype(jnp.bfloat16).astype(jnp.float32)
    yb2 = y2.astype(jnp.bfloat16).astype(jnp.float32)
    ab = (xb0 * yb0 + xb1 * yb1) + xb2 * yb2
    cur = (aa_i + aa_n) - 2.0 * ab

    m = jnp.full((R, 1), NEG, dtype=jnp.float32)
    for r in range(K + 1):
        mn = jnp.min(cur, axis=1, keepdims=True)
        mask = cur == mn
        if r >= 1:
            col = ncols[0, r - 1 : r, :]
            val = jnp.max(jnp.where(mask, col, NEG), axis=1, keepdims=True)
            m = jnp.maximum(m, val)
        cur = jnp.where(mask, POS, cur)

    out_ref[0, :, :C] = f
    out_ref[0, :, C:] = m - f


@jax.jit
def _run(feats2, coords2, coords_t2, ncols2):
    return pl.pallas_call(
        _body,
        grid=(2, NBLK),
        in_specs=[
            pl.BlockSpec((1, R, 3), lambda t, b: (t, b, 0)),
            pl.BlockSpec((1, 3, N), lambda t, b: (t, 0, 0)),
            pl.BlockSpec((1, K, N), lambda t, b: (t, 0, 0)),
            pl.BlockSpec((1, R, C), lambda t, b: (t, b, 0)),
        ],
        out_specs=pl.BlockSpec((1, R, 2 * C), lambda t, b: (t, b, 0)),
        out_shape=jax.ShapeDtypeStruct((2, N, 2 * C), jnp.float32),
    )(coords2, coords_t2, ncols2, feats2)


def kernel(src, tgt, src_coords, tgt_coords):
    feats2 = jnp.stack([src, tgt])
    coords2 = jnp.stack([src_coords, tgt_coords])
    coords_t2 = jnp.transpose(coords2, (0, 2, 1))
    ncols2 = jnp.transpose(feats2[:, :, :K], (0, 2, 1))
    out = _run(feats2, coords2, coords_t2, ncols2)
    return out[0], out[1]
